# baseline (speedup 1.0000x reference)
import math

import jax
import jax.numpy as jnp
from jax.experimental import pallas as pl
from jax.experimental.pallas import tpu as pltpu

_HEADS = 16
_HEADS_K = 4
_GROUP = _HEADS // _HEADS_K


def _qkv_proj_kernel(h_ref, wq_ref, wk_ref, wv_ref,
                     bq_ref, bk_ref, bv_ref, q_ref, k_ref, v_ref):
    x = h_ref[...]
    q_ref[...] = (jnp.dot(x, wq_ref[...], preferred_element_type=jnp.float32)
                  + bq_ref[...]).astype(jnp.bfloat16)
    k_ref[...] = (jnp.dot(x, wk_ref[...], preferred_element_type=jnp.float32)
                  + bk_ref[...]).astype(jnp.bfloat16)
    v_ref[...] = (jnp.dot(x, wv_ref[...], preferred_element_type=jnp.float32)
                  + bv_ref[...]).astype(jnp.bfloat16)


def _attn_kernel(q_ref, k_ref, v_ref, ao_ref):
    D = k_ref.shape[1] // _HEADS_K
    k = k_ref[...]
    v = v_ref[...]
    for h in range(_HEADS):
        hk = h // _GROUP
        q_h = q_ref[:, h * D:(h + 1) * D]
        k_h = k[:, hk * D:(hk + 1) * D]
        v_h = v[:, hk * D:(hk + 1) * D]
        s = jax.lax.dot_general(q_h, k_h, (((1,), (1,)), ((), ())),
                                preferred_element_type=jnp.float32)
        m = s.max(axis=-1, keepdims=True)
        p = jnp.exp(s - m)
        l = p.sum(axis=-1, keepdims=True)
        pv = jnp.dot(p, v_h, preferred_element_type=jnp.float32)
        ao_ref[:, h * D:(h + 1) * D] = (pv / l).astype(jnp.bfloat16)


def _out_proj_kernel(x_ref, w_ref, b_ref, o_ref):
    o_ref[...] = (jnp.dot(x_ref[...], w_ref[...],
                          preferred_element_type=jnp.float32) + b_ref[...])


def kernel(h, wq_t, bq, wk_t, bk, wv_t, bv, wo_t, bo):
    B, S, hidden = h.shape
    head_dim = hidden // _HEADS
    dkv = _HEADS_K * head_dim
    scale = 1.0 / math.sqrt(head_dim)
    M = B * S

    h2 = h.reshape(M, hidden)
    wq = wq_t * scale
    bq2 = (bq * scale).reshape(1, hidden)
    bk2 = bk.reshape(1, dkv)
    bv2 = bv.reshape(1, dkv)
    bo2 = bo.reshape(1, hidden)

    tm = 512
    q, k, v = pl.pallas_call(
        _qkv_proj_kernel,
        out_shape=(jax.ShapeDtypeStruct((M, hidden), jnp.bfloat16),
                   jax.ShapeDtypeStruct((M, dkv), jnp.bfloat16),
                   jax.ShapeDtypeStruct((M, dkv), jnp.bfloat16)),
        grid=(M // tm,),
        in_specs=[
            pl.BlockSpec((tm, hidden), lambda i: (i, 0)),
            pl.BlockSpec(memory_space=pltpu.VMEM),
            pl.BlockSpec(memory_space=pltpu.VMEM),
            pl.BlockSpec(memory_space=pltpu.VMEM),
            pl.BlockSpec(memory_space=pltpu.VMEM),
            pl.BlockSpec(memory_space=pltpu.VMEM),
            pl.BlockSpec(memory_space=pltpu.VMEM),
        ],
        out_specs=(pl.BlockSpec((tm, hidden), lambda i: (i, 0)),
                   pl.BlockSpec((tm, dkv), lambda i: (i, 0)),
                   pl.BlockSpec((tm, dkv), lambda i: (i, 0))),
        compiler_params=pltpu.CompilerParams(
            dimension_semantics=("parallel",),
            vmem_limit_bytes=60 * 1024 * 1024,
        ),
    )(h2, wq, wk_t, wv_t, bq2, bk2, bv2)

    ao = pl.pallas_call(
        _attn_kernel,
        out_shape=jax.ShapeDtypeStruct((M, hidden), jnp.bfloat16),
        grid=(B,),
        in_specs=[
            pl.BlockSpec((S, hidden), lambda i: (i, 0)),
            pl.BlockSpec((S, dkv), lambda i: (i, 0)),
            pl.BlockSpec((S, dkv), lambda i: (i, 0)),
        ],
        out_specs=pl.BlockSpec((S, hidden), lambda i: (i, 0)),
        compiler_params=pltpu.CompilerParams(
            dimension_semantics=("parallel",),
            vmem_limit_bytes=60 * 1024 * 1024,
        ),
    )(q, k, v)

    return pl.pallas_call(
        _out_proj_kernel,
        out_shape=jax.ShapeDtypeStruct((M, hidden), jnp.float32),
        grid=(M // tm,),
        in_specs=[
            pl.BlockSpec((tm, hidden), lambda i: (i, 0)),
            pl.BlockSpec(memory_space=pltpu.VMEM),
            pl.BlockSpec(memory_space=pltpu.VMEM),
        ],
        out_specs=pl.BlockSpec((tm, hidden), lambda i: (i, 0)),
        compiler_params=pltpu.CompilerParams(
            dimension_semantics=("parallel",),
            vmem_limit_bytes=60 * 1024 * 1024,
        ),
    )(ao, wo_t, bo2)


# speedup vs baseline: 1.0283x; 1.0283x over previous
import math

import jax
import jax.numpy as jnp
from jax.experimental import pallas as pl
from jax.experimental.pallas import tpu as pltpu

_HEADS = 16
_HEADS_K = 4
_GROUP = _HEADS // _HEADS_K


def _qkv_attn_kernel(h_ref, wq_ref, wk_ref, wv_ref,
                     bq_ref, bk_ref, bv_ref, ao_ref):
    D = wk_ref.shape[1] // _HEADS_K

    x = h_ref[...]
    q = jnp.dot(x, wq_ref[...], preferred_element_type=jnp.float32) + bq_ref[...]
    k = jnp.dot(x, wk_ref[...], preferred_element_type=jnp.float32) + bk_ref[...]
    v = jnp.dot(x, wv_ref[...], preferred_element_type=jnp.float32) + bv_ref[...]

    for h in range(_HEADS):
        hk = h // _GROUP
        q_h = q[:, h * D:(h + 1) * D]
        k_h = k[:, hk * D:(hk + 1) * D]
        v_h = v[:, hk * D:(hk + 1) * D]
        s = jax.lax.dot_general(q_h, k_h, (((1,), (1,)), ((), ())),
                                preferred_element_type=jnp.float32)
        m = s.max(axis=-1, keepdims=True)
        p = jnp.exp(s - m)
        l = p.sum(axis=-1, keepdims=True)
        pv = jnp.dot(p, v_h, preferred_element_type=jnp.float32)
        ao_ref[:, h * D:(h + 1) * D] = (pv / l).astype(jnp.bfloat16)


def _out_proj_kernel(x_ref, w_ref, b_ref, o_ref):
    o_ref[...] = (jnp.dot(x_ref[...], w_ref[...],
                          preferred_element_type=jnp.float32) + b_ref[...])


def kernel(h, wq_t, bq, wk_t, bk, wv_t, bv, wo_t, bo):
    B, S, hidden = h.shape
    head_dim = hidden // _HEADS
    dkv = _HEADS_K * head_dim
    scale = 1.0 / math.sqrt(head_dim)
    M = B * S

    h2 = h.reshape(M, hidden)
    wq = wq_t * scale
    bq2 = (bq * scale).reshape(1, hidden)
    bk2 = bk.reshape(1, dkv)
    bv2 = bv.reshape(1, dkv)
    bo2 = bo.reshape(1, hidden)

    ao = pl.pallas_call(
        _qkv_attn_kernel,
        out_shape=jax.ShapeDtypeStruct((M, hidden), jnp.bfloat16),
        grid=(B,),
        in_specs=[
            pl.BlockSpec((S, hidden), lambda i: (i, 0)),
            pl.BlockSpec(memory_space=pltpu.VMEM),
            pl.BlockSpec(memory_space=pltpu.VMEM),
            pl.BlockSpec(memory_space=pltpu.VMEM),
            pl.BlockSpec(memory_space=pltpu.VMEM),
            pl.BlockSpec(memory_space=pltpu.VMEM),
            pl.BlockSpec(memory_space=pltpu.VMEM),
        ],
        out_specs=pl.BlockSpec((S, hidden), lambda i: (i, 0)),
        compiler_params=pltpu.CompilerParams(
            dimension_semantics=("parallel",),
            vmem_limit_bytes=60 * 1024 * 1024,
        ),
    )(h2, wq, wk_t, wv_t, bq2, bk2, bv2)

    tm = 512
    return pl.pallas_call(
        _out_proj_kernel,
        out_shape=jax.ShapeDtypeStruct((M, hidden), jnp.float32),
        grid=(M // tm,),
        in_specs=[
            pl.BlockSpec((tm, hidden), lambda i: (i, 0)),
            pl.BlockSpec(memory_space=pltpu.VMEM),
            pl.BlockSpec(memory_space=pltpu.VMEM),
        ],
        out_specs=pl.BlockSpec((tm, hidden), lambda i: (i, 0)),
        compiler_params=pltpu.CompilerParams(
            dimension_semantics=("parallel",),
            vmem_limit_bytes=60 * 1024 * 1024,
        ),
    )(ao, wo_t, bo2)
